# baseline (speedup 1.0000x reference)
import functools

import jax
import jax.numpy as jnp
from jax.experimental import pallas as pl

_SIZES = (76, 38, 19)
_ANCHORS = {76: [[28, 28], [46, 45], [64, 66]],
            38: [[102, 74], [78, 115], [132, 113]],
            19: [[149, 163], [174, 268], [257, 176]]}
_ROWS = {76: 3 * 76 * 76, 38: 3 * 38 * 38, 19: 3 * 19 * 19}
_OFF = {76: 0, 38: _ROWS[76], 19: _ROWS[76] + _ROWS[38]}
_NBOX = _ROWS[76] + _ROWS[38] + _ROWS[19]


def _decode_tile(xa, s, aw, ah, y):
    stride = float(608 // s)
    t = xa.T
    en = jnp.exp(-t)
    sig = 1.0 / (1.0 + en)
    e = 1.0 / en
    c = jax.lax.broadcasted_iota(jnp.int32, (s, 85), 1)
    ix = jax.lax.broadcasted_iota(jnp.int32, (s, 85), 0).astype(jnp.float32)
    addx = (ix - 0.025) * stride
    addy = (y - 0.025) * stride
    add = jnp.where(c == 0, addx, addy)
    xyv = sig * (1.05 * stride) + add
    whv = e * jnp.where(c == 2, jnp.float32(aw), jnp.float32(ah))
    return jnp.where(c < 2, xyv, jnp.where(c < 4, whv, sig))


def _body(x0_ref, x1_ref, x2_ref, out_ref):
    g = pl.program_id(1)

    @pl.when(g < 10)
    def _scale76():
        for yy in range(8):
            y = g * 8 + yy
            yf = y.astype(jnp.float32)
            for a, (aw, ah) in enumerate(_ANCHORS[76]):
                res = _decode_tile(x0_ref[0, 85 * a:85 * (a + 1), yy, :],
                                   76, aw, ah, yf)
                start = y * 228 + a

                def _store(res=res, start=start):
                    out_ref[0:1, pl.Slice(start, 76, 3), :] = res[None]
                pl.when(y < 76)(_store)

    @pl.when(g == 10)
    def _scale38():
        for yy in range(38):
            for a, (aw, ah) in enumerate(_ANCHORS[38]):
                res = _decode_tile(x1_ref[0, 85 * a:85 * (a + 1), yy, :],
                                   38, aw, ah, float(yy))
                start = _OFF[38] + yy * 114 + a
                out_ref[0:1, pl.Slice(start, 38, 3), :] = res[None]

    @pl.when(g == 11)
    def _scale19():
        for yy in range(19):
            for a, (aw, ah) in enumerate(_ANCHORS[19]):
                res = _decode_tile(x2_ref[0, 85 * a:85 * (a + 1), yy, :],
                                   19, aw, ah, float(yy))
                start = _OFF[19] + yy * 57 + a
                out_ref[0:1, pl.Slice(start, 19, 3), :] = res[None]


def kernel(x0, x1, x2):
    b = x0.shape[0]
    return pl.pallas_call(
        _body,
        grid=(b, 12),
        in_specs=[
            pl.BlockSpec((1, 255, 8, 76),
                         lambda i, g: (i, 0, jnp.minimum(g, 9), 0)),
            pl.BlockSpec((1, 255, 38, 38), lambda i, g: (i, 0, 0, 0)),
            pl.BlockSpec((1, 255, 19, 19), lambda i, g: (i, 0, 0, 0)),
        ],
        out_specs=pl.BlockSpec((1, _NBOX, 85), lambda i, g: (i, 0, 0)),
        out_shape=jax.ShapeDtypeStruct((b, _NBOX, 85), jnp.float32),
    )(x0, x1, x2)


# speedup vs baseline: 1.0126x; 1.0126x over previous
import numpy as np
import jax
import jax.numpy as jnp
from jax.experimental import pallas as pl

_SIZES = (76, 38, 19)
_ANCHORS = {76: [[28, 28], [46, 45], [64, 66]],
            38: [[102, 74], [78, 115], [132, 113]],
            19: [[149, 163], [174, 268], [257, 176]]}
_POFF = {76: 0, 38: 76 * 76, 19: 76 * 76 + 38 * 38}
_NBOX = 3 * (76 * 76 + 38 * 38 + 19 * 19)


def _build_par(s: int) -> np.ndarray:
    n = s * s
    stride = float(608 // s)
    p = np.arange(n, dtype=np.float32)
    par = np.zeros((n, 4), dtype=np.float32)
    par[:, 0] = (np.mod(p, s) - 0.025) * stride
    par[:, 1] = (np.floor_divide(p, s) - 0.025) * stride
    par[:, 2] = 1.05 * stride
    return par


_PAR = {s: _build_par(s) for s in _SIZES}


def _body(x76, x38, x19, p76, p38, p19, out_ref):
    a = pl.program_id(1)
    for xr, pr, s in ((x76, p76, 76), (x38, p38, 38), (x19, p19, 19)):
        n = s * s
        t = xr[0, 0].T
        c = jax.lax.broadcasted_iota(jnp.int32, (n, 85), 1)
        sig = jax.nn.sigmoid(t)
        e = jnp.exp(t)
        add = jnp.where(c == 0, pr[:, 0:1], pr[:, 1:2])
        an = _ANCHORS[s]
        aw = jnp.where(a == 0, an[0][0], jnp.where(a == 1, an[1][0], an[2][0]))
        ah = jnp.where(a == 0, an[0][1], jnp.where(a == 1, an[1][1], an[2][1]))
        whv = e * jnp.where(c == 2, aw.astype(jnp.float32),
                            ah.astype(jnp.float32))
        res = jnp.where(c < 2, sig * pr[:, 2:3] + add,
                        jnp.where(c < 4, whv, sig))
        out_ref[0:1, pl.Slice(3 * _POFF[s] + a, n, 3), :] = res[None]


def kernel(x0, x1, x2):
    b = x0.shape[0]
    xs = [x.reshape(b, 3, 85, s * s)
          for x, s in zip((x0, x1, x2), _SIZES)]
    pars = [jnp.asarray(_PAR[s]) for s in _SIZES]

    def xspec(s):
        return pl.BlockSpec((1, 1, 85, s * s), lambda i, a: (i, a, 0, 0))

    def pspec(s):
        return pl.BlockSpec((s * s, 4), lambda i, a: (0, 0))

    return pl.pallas_call(
        _body,
        grid=(b, 3),
        in_specs=[xspec(s) for s in _SIZES] + [pspec(s) for s in _SIZES],
        out_specs=pl.BlockSpec((1, _NBOX, 85), lambda i, a: (i, 0, 0)),
        out_shape=jax.ShapeDtypeStruct((b, _NBOX, 85), jnp.float32),
    )(*xs, *pars)


# speedup vs baseline: 1.1301x; 1.1160x over previous
import numpy as np
import jax
import jax.numpy as jnp
from jax.experimental import pallas as pl

_SIZES = (76, 38, 19)
_ANCHORS = {76: [[28, 28], [46, 45], [64, 66]],
            38: [[102, 74], [78, 115], [132, 113]],
            19: [[149, 163], [174, 268], [257, 176]]}
_POFF = {76: 0, 38: 76 * 76, 19: 76 * 76 + 38 * 38}
_NBOX = 3 * (76 * 76 + 38 * 38 + 19 * 19)


def _build_add(s: int) -> np.ndarray:
    n = s * s
    stride = float(608 // s)
    p = np.arange(n, dtype=np.float32)
    add = np.zeros((2, n), dtype=np.float32)
    add[0] = (np.mod(p, s) - 0.025) * stride
    add[1] = (np.floor_divide(p, s) - 0.025) * stride
    return add


_ADD = {s: _build_add(s) for s in _SIZES}


def _body(x76, x38, x19, p76, p38, p19, out_ref):
    a = pl.program_id(1)
    for xr, pr, s in ((x76, p76, 76), (x38, p38, 38), (x19, p19, 19)):
        n = s * s
        m = 1.05 * float(608 // s)
        t = xr[0, 0]
        an = _ANCHORS[s]
        aw = jnp.where(a == 0, float(an[0][0]),
                       jnp.where(a == 1, float(an[1][0]), float(an[2][0])))
        ah = jnp.where(a == 0, float(an[0][1]),
                       jnp.where(a == 1, float(an[1][1]), float(an[2][1])))
        anc = jnp.concatenate([jnp.full((1, 1), aw, jnp.float32),
                               jnp.full((1, 1), ah, jnp.float32)], axis=0)
        xy = jax.nn.sigmoid(t[0:2, :]) * m + pr[...]
        wh = jnp.exp(t[2:4, :]) * anc
        cl = jax.nn.sigmoid(t[4:85, :])
        res = jnp.concatenate([xy, wh, cl], axis=0).T
        out_ref[0:1, pl.Slice(3 * _POFF[s] + a, n, 3), :] = res[None]


def kernel(x0, x1, x2):
    b = x0.shape[0]
    xs = [x.reshape(b, 3, 85, s * s)
          for x, s in zip((x0, x1, x2), _SIZES)]
    adds = [jnp.asarray(_ADD[s]) for s in _SIZES]

    def xspec(s):
        return pl.BlockSpec((1, 1, 85, s * s), lambda i, a: (i, a, 0, 0))

    def pspec(s):
        return pl.BlockSpec((2, s * s), lambda i, a: (0, 0))

    return pl.pallas_call(
        _body,
        grid=(b, 3),
        in_specs=[xspec(s) for s in _SIZES] + [pspec(s) for s in _SIZES],
        out_specs=pl.BlockSpec((1, _NBOX, 85), lambda i, a: (i, 0, 0)),
        out_shape=jax.ShapeDtypeStruct((b, _NBOX, 85), jnp.float32),
    )(*xs, *adds)
